# baseline (speedup 1.0000x reference)
import jax
import jax.numpy as jnp
from jax.experimental import pallas as pl
from jax.experimental.pallas import tpu as pltpu

_NSEM = 4


def kernel(sequence, positions):
    B, S, D = sequence.shape
    _, N = positions.shape
    pos = positions.astype(jnp.int32)

    def body(idx_ref, seq_ref, out_ref, *sems):
        b = pl.program_id(0)
        copies = []
        for r in range(N):
            idx = idx_ref[b, r]
            cp = pltpu.make_async_copy(
                seq_ref.at[b, pl.ds(idx, 1)],
                out_ref.at[0, pl.ds(r, 1)],
                sems[r % _NSEM],
            )
            cp.start()
            copies.append(cp)
        for cp in copies:
            cp.wait()

    return pl.pallas_call(
        body,
        grid_spec=pltpu.PrefetchScalarGridSpec(
            num_scalar_prefetch=1,
            grid=(B,),
            in_specs=[pl.BlockSpec(memory_space=pl.ANY)],
            out_specs=pl.BlockSpec((1, N, D), lambda b, idx_ref: (b, 0, 0)),
            scratch_shapes=[pltpu.SemaphoreType.DMA] * _NSEM,
        ),
        out_shape=jax.ShapeDtypeStruct((B, N, D), jnp.float32),
        compiler_params=pltpu.CompilerParams(
            dimension_semantics=("parallel",),
        ),
    )(pos, sequence)


# speedup vs baseline: 1.1028x; 1.1028x over previous
import jax
import jax.numpy as jnp
from jax.experimental import pallas as pl
from jax.experimental.pallas import tpu as pltpu


def kernel(sequence, positions):
    B, S, D = sequence.shape
    _, N = positions.shape
    pos = positions.astype(jnp.int32)

    def body(idx_ref, seq_ref, out_ref, sem):
        step = pl.program_id(0)
        copies = []
        for bb in range(2):
          b = step * 2 + bb
          for r in range(N):
            idx = idx_ref[b, r]
            cp = pltpu.make_async_copy(
                seq_ref.at[b, pl.ds(idx, 1)],
                out_ref.at[bb, pl.ds(r, 1)],
                sem,
            )
            cp.start()
            copies.append(cp)
        for cp in copies:
            cp.wait()

    return pl.pallas_call(
        body,
        grid_spec=pltpu.PrefetchScalarGridSpec(
            num_scalar_prefetch=1,
            grid=(B // 2,),
            in_specs=[pl.BlockSpec(memory_space=pl.ANY)],
            out_specs=pl.BlockSpec((2, N, D), lambda i, idx_ref: (i, 0, 0)),
            scratch_shapes=[pltpu.SemaphoreType.DMA],
        ),
        out_shape=jax.ShapeDtypeStruct((B, N, D), jnp.float32),
        compiler_params=pltpu.CompilerParams(
            dimension_semantics=("parallel",),
        ),
    )(pos, sequence)


# speedup vs baseline: 1.1202x; 1.0158x over previous
import jax
import jax.numpy as jnp
from jax.experimental import pallas as pl
from jax.experimental.pallas import tpu as pltpu


def kernel(sequence, positions):
    B, S, D = sequence.shape
    _, N = positions.shape
    pos = positions.astype(jnp.int32)

    def body(idx_ref, seq_ref, out_ref, sem):
        copies = []
        for b in range(B):
          for r in range(N):
            idx = idx_ref[b, r]
            cp = pltpu.make_async_copy(
                seq_ref.at[b, pl.ds(idx, 1)],
                out_ref.at[b, pl.ds(r, 1)],
                sem,
            )
            cp.start()
            copies.append(cp)
        for cp in copies:
            cp.wait()

    return pl.pallas_call(
        body,
        grid_spec=pltpu.PrefetchScalarGridSpec(
            num_scalar_prefetch=1,
            grid=(1,),
            in_specs=[pl.BlockSpec(memory_space=pl.ANY)],
            out_specs=pl.BlockSpec((B, N, D), lambda i, idx_ref: (0, 0, 0)),
            scratch_shapes=[pltpu.SemaphoreType.DMA],
        ),
        out_shape=jax.ShapeDtypeStruct((B, N, D), jnp.float32),
        compiler_params=pltpu.CompilerParams(
            dimension_semantics=("parallel",),
        ),
    )(pos, sequence)


# speedup vs baseline: 1.1669x; 1.0417x over previous
import jax
import jax.numpy as jnp
from jax.experimental import pallas as pl
from jax.experimental.pallas import tpu as pltpu


def kernel(sequence, positions):
    B, S, D = sequence.shape
    _, N = positions.shape
    pos = positions.astype(jnp.int32)

    def body(idx_ref, seq_ref, out_ref, scratch, wsem, *gsems):
        gathers = [[] for _ in range(B)]
        for b in range(B):
            for r in range(N):
                idx = idx_ref[b, r]
                cp = pltpu.make_async_copy(
                    seq_ref.at[b, pl.ds(idx, 1)],
                    scratch.at[b, pl.ds(r, 1)],
                    gsems[b],
                )
                cp.start()
                gathers[b].append(cp)
        writes = []
        for b in range(B):
            for cp in gathers[b]:
                cp.wait()
            wr = pltpu.make_async_copy(
                scratch.at[b], out_ref.at[b], wsem,
            )
            wr.start()
            writes.append(wr)
        for wr in writes:
            wr.wait()

    return pl.pallas_call(
        body,
        grid_spec=pltpu.PrefetchScalarGridSpec(
            num_scalar_prefetch=1,
            grid=(1,),
            in_specs=[pl.BlockSpec(memory_space=pl.ANY)],
            out_specs=pl.BlockSpec(memory_space=pl.ANY),
            scratch_shapes=[
                pltpu.VMEM((B, N, D), jnp.float32),
                pltpu.SemaphoreType.DMA,
            ] + [pltpu.SemaphoreType.DMA] * B,
        ),
        out_shape=jax.ShapeDtypeStruct((B, N, D), jnp.float32),
    )(pos, sequence)
